# Initial kernel scaffold; baseline (speedup 1.0000x reference)
import jax
import jax.numpy as jnp
from jax.experimental import pallas as pl


def kernel(inputs, embeddings):
    raise NotImplementedError("write your pallas kernel here")


# speedup vs baseline: 1.1087x; 1.1087x over previous
import functools

import jax
import jax.numpy as jnp
from jax import lax
from jax.experimental import pallas as pl
from jax.experimental.pallas import tpu as pltpu
from jax.experimental.pallas import tpu_sc as plsc

EMB = 32

_INFO = plsc.get_sparse_core_info()
NC = _INFO.num_cores
NS = _INFO.num_subcores
NW = NC * NS

CHUNK = 1600


def _gather_body(n_chunks, idx_hbm, table_hbm, out_hbm, idx_v0, idx_v1,
                 rows_v, gsem, osem):
  c = lax.axis_index("c")
  s = lax.axis_index("s")
  wid = s * NC + c
  b_per_w = n_chunks * CHUNK
  base = wid * b_per_w
  idx_bufs = [idx_v0, idx_v1]

  pltpu.sync_copy(idx_hbm.at[pl.ds(base, CHUNK)], idx_bufs[0])
  g = pltpu.async_copy(table_hbm.at[idx_bufs[0]], rows_v.at[0], gsem)
  out_pending = [None, None]
  for i in range(n_chunks):
    cur = i % 2
    nxt = 1 - cur
    if i + 1 < n_chunks:
      pltpu.sync_copy(idx_hbm.at[pl.ds(base + (i + 1) * CHUNK, CHUNK)],
                      idx_bufs[nxt])
    g.wait()
    if i + 1 < n_chunks:
      if out_pending[nxt] is not None:
        out_pending[nxt].wait()
        out_pending[nxt] = None
      g = pltpu.async_copy(table_hbm.at[idx_bufs[nxt]], rows_v.at[nxt], gsem)
    out_pending[cur] = pltpu.async_copy(
        rows_v.at[cur], out_hbm.at[pl.ds(base + i * CHUNK, CHUNK)], osem)
  for p in out_pending:
    if p is not None:
      p.wait()


def kernel(inputs, embeddings):
  orig_shape = inputs.shape
  flat_idx = jnp.reshape(inputs, (-1,)).astype(jnp.int32)
  b = flat_idx.shape[0]
  assert b % (NW * CHUNK) == 0, (b, NW * CHUNK)
  n_chunks = b // (NW * CHUNK)

  mesh = plsc.VectorSubcoreMesh(core_axis_name="c", subcore_axis_name="s")
  gather = pl.kernel(
      functools.partial(_gather_body, n_chunks),
      out_type=jax.ShapeDtypeStruct((b, EMB), jnp.float32),
      mesh=mesh,
      scratch_types=[
          pltpu.VMEM((CHUNK,), jnp.int32),
          pltpu.VMEM((CHUNK,), jnp.int32),
          pltpu.VMEM((2, CHUNK, EMB), jnp.float32),
          pltpu.SemaphoreType.DMA,
          pltpu.SemaphoreType.DMA,
      ],
      compiler_params=pltpu.CompilerParams(use_tc_tiling_on_sc=False),
  )
  out = gather(flat_idx, embeddings)
  return jnp.reshape(out, orig_shape + (EMB,))


# speedup vs baseline: 1.7907x; 1.6150x over previous
import functools

import jax
import jax.numpy as jnp
from jax import lax
from jax.experimental import pallas as pl
from jax.experimental.pallas import tpu as pltpu
from jax.experimental.pallas import tpu_sc as plsc

EMB = 32
SEQ = 50

_INFO = plsc.get_sparse_core_info()
NC = _INFO.num_cores
NS = _INFO.num_subcores
NW = NC * NS

CHUNK_R = 32


def _gather_body(n_chunks, idx_hbm, table_hbm, out_hbm, idx_v0, idx_v1,
                 rows_v0, rows_v1, gsem, osem):
  c = lax.axis_index("c")
  s = lax.axis_index("s")
  wid = s * NC + c
  chunk_f = CHUNK_R * SEQ
  base_r = wid * n_chunks * CHUNK_R
  base_f = base_r * SEQ
  idx_bufs = [idx_v0, idx_v1]
  rows_bufs = [rows_v0, rows_v1]

  pltpu.sync_copy(idx_hbm.at[pl.ds(base_f, chunk_f)], idx_bufs[0])
  g = pltpu.async_copy(table_hbm.at[idx_bufs[0]], rows_bufs[0], gsem)
  out_pending = [[], []]
  for i in range(n_chunks):
    cur = i % 2
    nxt = 1 - cur
    if i + 1 < n_chunks:
      pltpu.sync_copy(
          idx_hbm.at[pl.ds(base_f + (i + 1) * chunk_f, chunk_f)],
          idx_bufs[nxt])
    g.wait()
    if i + 1 < n_chunks:
      for p in out_pending[nxt]:
        p.wait()
      out_pending[nxt] = []
      g = pltpu.async_copy(table_hbm.at[idx_bufs[nxt]], rows_bufs[nxt], gsem)
    out_pending[cur] = [
        pltpu.async_copy(rows_bufs[cur].at[pl.ds(r * SEQ, SEQ)],
                         out_hbm.at[base_r + i * CHUNK_R + r], osem)
        for r in range(CHUNK_R)
    ]
  for plist in out_pending:
    for p in plist:
      p.wait()


def kernel(inputs, embeddings):
  b, seq = inputs.shape
  flat_idx = jnp.reshape(inputs, (-1,)).astype(jnp.int32)
  assert seq == SEQ and b % (NW * CHUNK_R) == 0, (inputs.shape,)
  n_chunks = b // (NW * CHUNK_R)

  mesh = plsc.VectorSubcoreMesh(core_axis_name="c", subcore_axis_name="s")
  gather = pl.kernel(
      functools.partial(_gather_body, n_chunks),
      out_type=jax.ShapeDtypeStruct((b, SEQ, EMB), jnp.float32),
      mesh=mesh,
      scratch_types=[
          pltpu.VMEM((CHUNK_R * SEQ,), jnp.int32),
          pltpu.VMEM((CHUNK_R * SEQ,), jnp.int32),
          pltpu.VMEM((CHUNK_R * SEQ, EMB), jnp.float32),
          pltpu.VMEM((CHUNK_R * SEQ, EMB), jnp.float32),
          pltpu.SemaphoreType.DMA,
          pltpu.SemaphoreType.DMA,
      ],
      compiler_params=pltpu.CompilerParams(use_tc_tiling_on_sc=False),
  )
  return gather(flat_idx, embeddings)
